# baseline (speedup 1.0000x reference)
import jax
import jax.numpy as jnp
from jax import lax
from jax.experimental import pallas as pl
from jax.experimental.pallas import tpu as pltpu
from jax.experimental.pallas import tpu_sc as plsc

GRID = 128
N = 100000
CIN = 16
COUT = 16
K = 3

NC = 2
NS = 16
NW = NC * NS
B = 128
CHUNK = 3200
NP = NW * CHUNK
NTAP = 27

GDIM = GRID + 2
GCELLS_USED = GDIM ** 3
INIT_W = 2048
INIT_REPS = 68
GCELLS = NS * INIT_REPS * INIT_W

OFFS = [(dz, dy, dx) for dz in range(K) for dy in range(K) for dx in range(K)]
DELTA = [
    (dz - 1) * GDIM * GDIM + (dy - 1) * GDIM + (dx - 1)
    for dz, dy, dx in OFFS
]


def _sc_gather_kernel(flat_hbm, vals_hbm, feats_hbm, g_hbm,
                      grid_hbm, init_v, flat_v, val_v, adrx_v, ridx_v,
                      rows_v):
    cid = lax.axis_index("c")
    sid = lax.axis_index("s")
    wid = cid * NS + sid
    coff = cid * GCELLS

    @pl.loop(0, INIT_W, step=16)
    def _(i):
        init_v[pl.ds(i, 16)] = jnp.full((16,), N, jnp.int32)

    tile_words = INIT_REPS * INIT_W

    @pl.loop(0, tile_words, step=INIT_W)
    def _(j):
        pltpu.sync_copy(init_v,
                        grid_hbm.at[pl.ds(coff + sid * tile_words + j,
                                          INIT_W)])

    plsc.subcore_barrier()

    scat_per_tile = NP // NS

    @pl.loop(0, scat_per_tile, step=B)
    def _(i):
        base = sid * scat_per_tile + i
        pltpu.sync_copy(flat_hbm.at[pl.ds(base, B)], flat_v)
        pltpu.sync_copy(vals_hbm.at[pl.ds(base, B)], val_v)
        for j in range(B // 16):
            sl = pl.ds(j * 16, 16)
            flat_v[sl] = flat_v[sl] + coff
        pltpu.sync_copy(val_v, grid_hbm.at[flat_v])

    plsc.subcore_barrier()

    iota16 = lax.iota(jnp.int32, 16)
    t27 = iota16 * NTAP

    @pl.loop(0, CHUNK, step=B)
    def _(bstart):
        vstart = wid * CHUNK + bstart
        pltpu.sync_copy(flat_hbm.at[pl.ds(vstart, B)], flat_v)

        for j in range(B // 16):
            f = flat_v[pl.ds(j * 16, 16)] + coff
            for o in range(NTAP):
                pos = t27 + (j * 16 * NTAP + o)
                prow = lax.shift_right_logical(pos, 7)
                pcol = lax.bitwise_and(pos, 127)
                plsc.store_scatter(adrx_v, [prow, pcol], f + DELTA[o])

        @pl.loop(0, NTAP)
        def _(o):
            pltpu.sync_copy(grid_hbm.at[adrx_v.at[o]], ridx_v.at[o])

        @pl.loop(0, NTAP)
        def _(o):
            pltpu.sync_copy(feats_hbm.at[ridx_v.at[o]],
                            rows_v.at[pl.ds(o * B, B)])

        pltpu.sync_copy(rows_v, g_hbm.at[pl.ds(vstart * NTAP, B * NTAP), :])


def _tc_matmul_kernel(g_ref, w_ref, b_ref, o_ref):
    acc = jnp.dot(g_ref[...], w_ref[...], preferred_element_type=jnp.float32)
    o_ref[...] = acc + b_ref[...]


@jax.jit
def kernel(feats, coords, weight, bias):
    n = feats.shape[0]
    fg = (((coords[:, 0] + 1) * GDIM + (coords[:, 1] + 1)) * GDIM
          + (coords[:, 2] + 1)).astype(jnp.int32)
    flat_pad = jnp.concatenate(
        [fg, jnp.broadcast_to(fg[0], (NP - n,))])
    vals = jnp.concatenate(
        [jnp.arange(n, dtype=jnp.int32), jnp.zeros((NP - n,), jnp.int32)])
    feats_pad = jnp.zeros((NP, CIN), jnp.float32).at[:n].set(feats)

    mesh = plsc.VectorSubcoreMesh(core_axis_name="c", subcore_axis_name="s")
    sc = pl.kernel(
        _sc_gather_kernel,
        out_type=jax.ShapeDtypeStruct((NP * NTAP, CIN), jnp.float32),
        mesh=mesh,
        compiler_params=pltpu.CompilerParams(needs_layout_passes=False,
                                             use_tc_tiling_on_sc=False),
        scratch_types=[
            pltpu.HBM((NC * GCELLS,), jnp.int32),
            pltpu.VMEM((INIT_W,), jnp.int32),
            pltpu.VMEM((B,), jnp.int32),
            pltpu.VMEM((B,), jnp.int32),
            pltpu.VMEM((NTAP, B), jnp.int32),
            pltpu.VMEM((NTAP, B), jnp.int32),
            pltpu.VMEM((NTAP * B, CIN), jnp.float32),
        ],
    )
    g = sc(flat_pad, vals, feats_pad)
    g2 = g.reshape(NP, NTAP * CIN)

    w2 = jnp.transpose(weight, (1, 2, 3, 4, 0)).reshape(NTAP * CIN, COUT)
    b2 = bias.reshape(1, COUT)

    bn = 2000
    out = pl.pallas_call(
        _tc_matmul_kernel,
        grid=(n // bn,),
        in_specs=[
            pl.BlockSpec((bn, NTAP * CIN), lambda i: (i, 0)),
            pl.BlockSpec((NTAP * CIN, COUT), lambda i: (0, 0)),
            pl.BlockSpec((1, COUT), lambda i: (0, 0)),
        ],
        out_specs=pl.BlockSpec((bn, COUT), lambda i: (i, 0)),
        out_shape=jax.ShapeDtypeStruct((n, COUT), jnp.float32),
    )(g2, w2, b2)
    return out


# speedup vs baseline: 1.0005x; 1.0005x over previous
import jax
import jax.numpy as jnp
from jax import lax
from jax.experimental import pallas as pl
from jax.experimental.pallas import tpu as pltpu
from jax.experimental.pallas import tpu_sc as plsc

GRID = 128
N = 100000
CIN = 16
COUT = 16
K = 3

NC = 2
NS = 16
NW = NC * NS
B = 128
CHUNK = 3200
NP = NW * CHUNK
NROWS = NP // B
NTAP = 27

GDIM = GRID + 2
INIT_W = 8192
INIT_REPS = 17
GCELLS = NS * INIT_REPS * INIT_W

SCAT_R = 5

OFFS = [(dz, dy, dx) for dz in range(K) for dy in range(K) for dx in range(K)]
DELTA = [
    (dz - 1) * GDIM * GDIM + (dy - 1) * GDIM + (dx - 1)
    for dz, dy, dx in OFFS
]


def _sc_gather_kernel(flat_hbm, vals_hbm, feats_hbm, g_hbm,
                      grid_hbm, init_v, flatb_v, valb_v, flatc_v,
                      adrx_v, ridx_v, rows_v,
                      semi, seml, sems, semc, semd, seme):
    cid = lax.axis_index("c")
    sid = lax.axis_index("s")
    wid = cid * NS + sid
    coff = cid * GCELLS

    @pl.loop(0, INIT_W, step=16)
    def _(i):
        init_v[pl.ds(i, 16)] = jnp.full((16,), N, jnp.int32)

    tile_words = INIT_REPS * INIT_W

    @pl.loop(0, INIT_REPS)
    def _(j):
        pltpu.async_copy(
            init_v,
            grid_hbm.at[pl.ds(coff + sid * tile_words + j * INIT_W, INIT_W)],
            semi)

    @pl.loop(0, INIT_REPS)
    def _(j):
        pltpu.make_async_copy(
            init_v,
            grid_hbm.at[pl.ds(coff + sid * tile_words + j * INIT_W, INIT_W)],
            semi).wait()

    plsc.subcore_barrier()

    rows_per_tile = NROWS // NS

    @pl.loop(0, rows_per_tile, step=SCAT_R)
    def _(i):
        base = sid * rows_per_tile + i
        pltpu.async_copy(flat_hbm.at[pl.ds(base, SCAT_R)], flatb_v, seml)
        pltpu.async_copy(vals_hbm.at[pl.ds(base, SCAT_R)], valb_v, seml)
        pltpu.make_async_copy(flat_hbm.at[pl.ds(base, SCAT_R)], flatb_v,
                              seml).wait()
        pltpu.make_async_copy(vals_hbm.at[pl.ds(base, SCAT_R)], valb_v,
                              seml).wait()
        for r in range(SCAT_R):
            for j in range(B // 16):
                sl = pl.ds(j * 16, 16)
                flatb_v[r, sl] = flatb_v[r, sl] + coff
        for r in range(SCAT_R):
            pltpu.async_copy(valb_v.at[r], grid_hbm.at[flatb_v.at[r]], sems)
        for r in range(SCAT_R):
            pltpu.make_async_copy(valb_v.at[r], grid_hbm.at[flatb_v.at[r]],
                                  sems).wait()

    plsc.subcore_barrier()

    iota16 = lax.iota(jnp.int32, 16)
    t27 = iota16 * NTAP

    def fire_e(vstart):
        return pltpu.make_async_copy(
            rows_v, g_hbm.at[pl.ds(vstart * NTAP, B * NTAP), :], seme)

    @pl.loop(0, CHUNK, step=B)
    def _(bstart):
        vstart = wid * CHUNK + bstart
        vrow = vstart // B
        pltpu.sync_copy(flat_hbm.at[pl.ds(vrow, 1)], flatc_v)

        for j in range(B // 16):
            f = flatc_v[0, pl.ds(j * 16, 16)] + coff
            for o in range(NTAP):
                pos = t27 + (j * 16 * NTAP + o)
                prow = lax.shift_right_logical(pos, 7)
                pcol = lax.bitwise_and(pos, 127)
                plsc.store_scatter(adrx_v, [prow, pcol], f + DELTA[o])

        @pl.loop(0, NTAP)
        def _(o):
            pltpu.async_copy(grid_hbm.at[adrx_v.at[o]], ridx_v.at[o], semc)

        @pl.loop(0, NTAP)
        def _(o):
            pltpu.make_async_copy(grid_hbm.at[adrx_v.at[o]], ridx_v.at[o],
                                  semc).wait()

        @pl.when(bstart > 0)
        def _():
            fire_e(vstart - B).wait()

        @pl.loop(0, NTAP)
        def _(o):
            pltpu.async_copy(feats_hbm.at[ridx_v.at[o]],
                             rows_v.at[pl.ds(o * B, B)], semd)

        @pl.loop(0, NTAP)
        def _(o):
            pltpu.make_async_copy(feats_hbm.at[ridx_v.at[o]],
                                  rows_v.at[pl.ds(o * B, B)], semd).wait()

        pltpu.async_copy(rows_v, g_hbm.at[pl.ds(vstart * NTAP, B * NTAP), :],
                         seme)

    fire_e(wid * CHUNK + CHUNK - B).wait()


def _tc_matmul_kernel(g_ref, w_ref, b_ref, o_ref):
    acc = jnp.dot(g_ref[...], w_ref[...], preferred_element_type=jnp.float32)
    o_ref[...] = acc + b_ref[...]


@jax.jit
def kernel(feats, coords, weight, bias):
    n = feats.shape[0]
    fg = (((coords[:, 0] + 1) * GDIM + (coords[:, 1] + 1)) * GDIM
          + (coords[:, 2] + 1)).astype(jnp.int32)
    flat_pad = jnp.concatenate(
        [fg, jnp.broadcast_to(fg[0], (NP - n,))]).reshape(NROWS, B)
    vals = jnp.concatenate(
        [jnp.arange(n, dtype=jnp.int32),
         jnp.zeros((NP - n,), jnp.int32)]).reshape(NROWS, B)
    feats_pad = jnp.zeros((NP, CIN), jnp.float32).at[:n].set(feats)

    mesh = plsc.VectorSubcoreMesh(core_axis_name="c", subcore_axis_name="s")
    sc = pl.kernel(
        _sc_gather_kernel,
        out_type=jax.ShapeDtypeStruct((NP * NTAP, CIN), jnp.float32),
        mesh=mesh,
        compiler_params=pltpu.CompilerParams(needs_layout_passes=False,
                                             use_tc_tiling_on_sc=False),
        scratch_types=[
            pltpu.HBM((NC * GCELLS,), jnp.int32),
            pltpu.VMEM((INIT_W,), jnp.int32),
            pltpu.VMEM((SCAT_R, B), jnp.int32),
            pltpu.VMEM((SCAT_R, B), jnp.int32),
            pltpu.VMEM((1, B), jnp.int32),
            pltpu.VMEM((NTAP, B), jnp.int32),
            pltpu.VMEM((NTAP, B), jnp.int32),
            pltpu.VMEM((NTAP * B, CIN), jnp.float32),
            pltpu.SemaphoreType.DMA,
            pltpu.SemaphoreType.DMA,
            pltpu.SemaphoreType.DMA,
            pltpu.SemaphoreType.DMA,
            pltpu.SemaphoreType.DMA,
            pltpu.SemaphoreType.DMA,
        ],
    )
    g = sc(flat_pad, vals, feats_pad)
    g2 = g.reshape(NP, NTAP * CIN)

    w2 = jnp.transpose(weight, (1, 2, 3, 4, 0)).reshape(NTAP * CIN, COUT)
    b2 = bias.reshape(1, COUT)

    bn = 2000
    out = pl.pallas_call(
        _tc_matmul_kernel,
        grid=(n // bn,),
        in_specs=[
            pl.BlockSpec((bn, NTAP * CIN), lambda i: (i, 0)),
            pl.BlockSpec((NTAP * CIN, COUT), lambda i: (0, 0)),
            pl.BlockSpec((1, COUT), lambda i: (0, 0)),
        ],
        out_specs=pl.BlockSpec((bn, COUT), lambda i: (i, 0)),
        out_shape=jax.ShapeDtypeStruct((n, COUT), jnp.float32),
    )(g2, w2, b2)
    return out


# speedup vs baseline: 3.5964x; 3.5947x over previous
import functools

import jax
import jax.numpy as jnp
from jax import lax
from jax.experimental import pallas as pl
from jax.experimental.pallas import tpu as pltpu
from jax.experimental.pallas import tpu_sc as plsc

GRID = 128
N = 100000
CIN = 16
COUT = 16
K = 3

NC = 2
NS = 16
NW = NC * NS
CHUNK = 3200
NP = NW * CHUNK
NROWS = NP // 128
BV = 128
NBLK = CHUNK // BV
NT = 26

GDIM = GRID + 2
NCELL = GDIM ** 3
SAFE_CELL = 2197008
GSZ = 2197120
GSZ16 = GSZ // 16
CLAMPF = 2179968
NBW = 68864
NBW_TILE = NBW // NS
NCH = BV * NT // 128
SCAT_R = 5

OFFS = [(dz, dy, dx) for dz in range(K) for dy in range(K) for dx in range(K)
        if (dz, dy, dx) != (1, 1, 1)]
DELTA = [
    (dz - 1) * GDIM * GDIM + (dy - 1) * GDIM + (dx - 1) for dz, dy, dx in OFFS
]


def _sc_build_kernel(flat_hbm, vals_hbm, bitw_hbm, bitv_hbm,
                     grid_hbm, bmap_hbm,
                     bmap_sh, zero_v, flatb_v, valb_v, bitwb_v, bitvb_v,
                     semg):
    cid = lax.axis_index("c")
    sid = lax.axis_index("s")
    coff = cid * GSZ

    @pl.loop(0, NBW_TILE, step=16)
    def _(i):
        zero_v[pl.ds(i, 16)] = jnp.zeros((16,), jnp.int32)

    pltpu.sync_copy(zero_v, bmap_sh.at[pl.ds(sid * NBW_TILE, NBW_TILE)])
    plsc.subcore_barrier()

    rows_per_tile = NROWS // NS

    @pl.loop(0, rows_per_tile, step=SCAT_R)
    def _(i):
        base = sid * rows_per_tile + i
        pltpu.sync_copy(flat_hbm.at[pl.ds(base, SCAT_R)], flatb_v)
        pltpu.sync_copy(vals_hbm.at[pl.ds(base, SCAT_R)], valb_v)
        pltpu.sync_copy(bitw_hbm.at[pl.ds(base, SCAT_R)], bitwb_v)
        pltpu.sync_copy(bitv_hbm.at[pl.ds(base, SCAT_R)], bitvb_v)
        for r in range(SCAT_R):
            for j in range(8):
                sl = pl.ds(j * 16, 16)
                flatb_v[r, sl] = flatb_v[r, sl] + coff
        for r in range(SCAT_R):
            pltpu.async_copy(valb_v.at[r], grid_hbm.at[flatb_v.at[r]], semg)
            pltpu.sync_copy(bitvb_v.at[r], bmap_sh.at[bitwb_v.at[r]],
                            add=True)
        for r in range(SCAT_R):
            pltpu.make_async_copy(valb_v.at[r], grid_hbm.at[flatb_v.at[r]],
                                  semg).wait()

    plsc.subcore_barrier()

    @pl.when(cid == 0)
    def _():
        pltpu.sync_copy(bmap_sh.at[pl.ds(sid * NBW_TILE, NBW_TILE)],
                        bmap_hbm.at[pl.ds(sid * NBW_TILE, NBW_TILE)])


def _sc_gather_kernel(flat_hbm, bmap_hbm, grid2_hbm, feats_hbm,
                      g_hbm, presw_hbm,
                      bmap_v, flatc_v, cadr_v, cdst_v, rowg_v, crid2_v,
                      cdst2_v, rowsb_v, presb_v):
    cid = lax.axis_index("c")
    sid = lax.axis_index("s")
    wid = cid * NS + sid
    coff16 = cid * GSZ16

    pltpu.sync_copy(bmap_hbm, bmap_v)

    iota16 = lax.iota(jnp.int32, 16)
    one16 = jnp.full((16,), 1, jnp.int32)
    i16x16 = iota16 * 16
    clampf = jnp.full((16,), CLAMPF, jnp.int32)

    @pl.loop(0, NBLK)
    def _(blk):
        vstart = wid * CHUNK + blk * BV
        vrow = vstart // BV
        pltpu.sync_copy(flat_hbm.at[pl.ds(vrow, 1)], flatc_v)

        off = jnp.int32(0)
        vs26 = vstart * NT
        for j in range(BV // 16):
            f = jnp.minimum(flatc_v[0, pl.ds(j * 16, 16)], clampf)
            pw = jnp.zeros((16,), jnp.int32)
            for oi in range(NT):
                adr = f + DELTA[oi]
                w = plsc.load_gather(bmap_v,
                                     [lax.shift_right_logical(adr, 5)])
                bit = lax.bitwise_and(
                    lax.shift_right_logical(w, lax.bitwise_and(adr, 31)),
                    one16)
                m = bit > 0
                pw = pw + lax.shift_left(bit, oi)
                ps = plsc.cumsum(bit)
                pos = (ps - bit) + off
                prow = lax.shift_right_logical(pos, 7)
                pcol = lax.bitwise_and(pos, 127)
                row16 = lax.shift_right_logical(adr, 4) + coff16
                dst4 = t26d4(vs26, j, oi, iota16) + lax.bitwise_and(adr, 15)
                plsc.store_scatter(cadr_v, [prow, pcol], row16, mask=m)
                plsc.store_scatter(cdst_v, [prow, pcol], dst4, mask=m)
                off = off + jnp.max(ps)
            presb_v[0, pl.ds(j * 16, 16)] = pw

        pltpu.sync_copy(presb_v, presw_hbm.at[pl.ds(vrow, 1)])

        lastrow = lax.shift_right_logical(off, 7)
        fillstart = lax.bitwise_and(off, 127)
        saferow = jnp.full((16,), SAFE_CELL // 16, jnp.int32) + coff16
        for j in range(8):
            lane = iota16 + (j * 16)
            fm = lane >= fillstart
            lr = jnp.broadcast_to(lastrow, (16,))
            plsc.store_scatter(cadr_v, [lr, lane], saferow, mask=fm)
            plsc.store_scatter(
                cdst_v, [lr, lane],
                lax.shift_left(jnp.full((16,), NP * NT, jnp.int32) + iota16,
                               4), mask=fm)

        nch = lax.shift_right_logical(off + 127, 7)

        @pl.loop(0, NCH)
        def _(t):
            @pl.when(t < nch)
            def _():
                pltpu.sync_copy(grid2_hbm.at[cadr_v.at[t]], rowg_v)
                ts = jnp.broadcast_to(t, (16,))
                for v in range(8):
                    sl = pl.ds(v * 16, 16)
                    kv = iota16 + (v * 16)
                    d4 = plsc.load_gather(cdst_v, [ts, kv])
                    lan = lax.bitwise_and(d4, 15)
                    rid = plsc.load_gather(rowg_v, [kv, lan])
                    crid2_v[sl] = rid
                    cdst2_v[sl] = lax.shift_right_logical(d4, 4)
                pltpu.sync_copy(feats_hbm.at[crid2_v], rowsb_v)
                pltpu.sync_copy(rowsb_v, g_hbm.at[cdst2_v])


def t26d4(vs26, j, oi, iota16):
    return lax.shift_left(iota16 * NT + (vs26 + j * 16 * NT + oi), 4)


def _tc_kernel(g_ref, feats_ref, presw_ref, w26_ref, wc_ref, b_ref, o_ref,
               *, bn):
    g = g_ref[...]
    pw = presw_ref[...]
    lane = lax.broadcasted_iota(jnp.int32, (1, NT * CIN), 1)
    sh = lane // CIN
    msk = jnp.right_shift(pw, sh) & 1
    gm = g * msk.astype(jnp.float32)
    acc = jnp.dot(gm, w26_ref[...], preferred_element_type=jnp.float32)
    acc = acc + jnp.dot(feats_ref[...], wc_ref[...],
                        preferred_element_type=jnp.float32)
    o_ref[...] = acc + b_ref[...]


@jax.jit
def kernel(feats, coords, weight, bias):
    n = feats.shape[0]
    fg = (((coords[:, 0] + 1) * GDIM + (coords[:, 1] + 1)) * GDIM
          + (coords[:, 2] + 1)).astype(jnp.int32)
    pad_flat = jnp.broadcast_to(fg[0], (NP - n,))
    pad_flat = pad_flat.at[0].set(SAFE_CELL)
    flat_pad = jnp.concatenate([fg, pad_flat]).reshape(NROWS, 128)
    vals = jnp.concatenate(
        [jnp.arange(n, dtype=jnp.int32),
         jnp.zeros((NP - n,), jnp.int32)]).reshape(NROWS, 128)
    bitw = flat_pad >> 5
    bitv = jnp.where(
        (jnp.arange(NP, dtype=jnp.int32) < n).reshape(NROWS, 128),
        jnp.int32(1) << (flat_pad & 31), 0)

    mesh = plsc.VectorSubcoreMesh(core_axis_name="c", subcore_axis_name="s")
    cp = pltpu.CompilerParams(needs_layout_passes=False,
                              use_tc_tiling_on_sc=False)
    build = pl.kernel(
        _sc_build_kernel,
        out_type=(
            jax.ShapeDtypeStruct((NC * GSZ,), jnp.int32),
            jax.ShapeDtypeStruct((NBW,), jnp.int32),
        ),
        mesh=mesh,
        compiler_params=cp,
        scratch_types=[
            pltpu.VMEM_SHARED((NBW,), jnp.int32),
            pltpu.VMEM((NBW_TILE,), jnp.int32),
            pltpu.VMEM((SCAT_R, 128), jnp.int32),
            pltpu.VMEM((SCAT_R, 128), jnp.int32),
            pltpu.VMEM((SCAT_R, 128), jnp.int32),
            pltpu.VMEM((SCAT_R, 128), jnp.int32),
            pltpu.SemaphoreType.DMA,
        ],
    )
    grid1d, bmap = build(flat_pad, vals, bitw, bitv)
    grid2 = grid1d.reshape(NC * GSZ16, 16)

    gather = pl.kernel(
        _sc_gather_kernel,
        out_type=(
            jax.ShapeDtypeStruct((NP * NT + 128, CIN), jnp.float32),
            jax.ShapeDtypeStruct((NROWS, 128), jnp.int32),
        ),
        mesh=mesh,
        compiler_params=cp,
        scratch_types=[
            pltpu.VMEM((NBW,), jnp.int32),
            pltpu.VMEM((1, 128), jnp.int32),
            pltpu.VMEM((NCH + 1, 128), jnp.int32),
            pltpu.VMEM((NCH + 1, 128), jnp.int32),
            pltpu.VMEM((128, 16), jnp.int32),
            pltpu.VMEM((128,), jnp.int32),
            pltpu.VMEM((128,), jnp.int32),
            pltpu.VMEM((128, CIN), jnp.float32),
            pltpu.VMEM((1, 128), jnp.int32),
        ],
    )
    g, presw = gather(flat_pad, bmap, grid2, feats)

    g3 = g[:NP * NT].reshape(NP, NT * CIN)
    presw2 = presw.reshape(NP, 1)
    feats_pad = jnp.zeros((NP, CIN), jnp.float32).at[:n].set(feats)

    w = jnp.transpose(weight, (1, 2, 3, 4, 0)).reshape(27, CIN, COUT)
    keep = [i for i in range(27) if i != 13]
    w26 = w[jnp.array(keep)].reshape(NT * CIN, COUT)
    wc = w[13]
    b2 = bias.reshape(1, COUT)

    bn = 2048
    out = pl.pallas_call(
        functools.partial(_tc_kernel, bn=bn),
        grid=(NP // bn,),
        in_specs=[
            pl.BlockSpec((bn, NT * CIN), lambda i: (i, 0)),
            pl.BlockSpec((bn, CIN), lambda i: (i, 0)),
            pl.BlockSpec((bn, 1), lambda i: (i, 0)),
            pl.BlockSpec((NT * CIN, COUT), lambda i: (0, 0)),
            pl.BlockSpec((CIN, COUT), lambda i: (0, 0)),
            pl.BlockSpec((1, COUT), lambda i: (0, 0)),
        ],
        out_specs=pl.BlockSpec((bn, COUT), lambda i: (i, 0)),
        out_shape=jax.ShapeDtypeStruct((NP, COUT), jnp.float32),
    )(g3, feats_pad, presw2, w26, wc, b2)
    return out[:n]


# speedup vs baseline: 9.2563x; 2.5738x over previous
import functools

import jax
import jax.numpy as jnp
from jax import lax
from jax.experimental import pallas as pl
from jax.experimental.pallas import tpu as pltpu
from jax.experimental.pallas import tpu_sc as plsc

GRID = 128
N = 100000
CIN = 16
COUT = 16
K = 3

NC = 2
NS = 16
NW = NC * NS
CHUNK = 3200
NP = NW * CHUNK
NROWS = NP // 128
BV = 128
NBLK = CHUNK // BV
NT = 26

GDIM = GRID + 2
NCELL = GDIM ** 3
SAFE_CELL = 2197008
GSZ = 2197120
GSZ16 = GSZ // 16
CLAMPF = 2179968
NBW = 68864
NBW_TILE = NBW // NS
NCH = BV * NT // 128
SCAT_R = 5

OFFS = [(dz, dy, dx) for dz in range(K) for dy in range(K) for dx in range(K)
        if (dz, dy, dx) != (1, 1, 1)]
DELTA = [
    (dz - 1) * GDIM * GDIM + (dy - 1) * GDIM + (dx - 1) for dz, dy, dx in OFFS
]


def _sc_build_kernel(flat_hbm, vals_hbm, bitw_hbm, bitv_hbm,
                     grid_hbm, bmap_hbm,
                     bmap_sh, zero_v, flatb_v, valb_v, bitwb_v, bitvb_v,
                     semg):
    cid = lax.axis_index("c")
    sid = lax.axis_index("s")
    coff = cid * GSZ

    @pl.loop(0, NBW_TILE, step=16)
    def _(i):
        zero_v[pl.ds(i, 16)] = jnp.zeros((16,), jnp.int32)

    pltpu.sync_copy(zero_v, bmap_sh.at[pl.ds(sid * NBW_TILE, NBW_TILE)])
    plsc.subcore_barrier()

    rows_per_tile = NROWS // NS

    @pl.loop(0, rows_per_tile, step=SCAT_R)
    def _(i):
        base = sid * rows_per_tile + i
        pltpu.sync_copy(flat_hbm.at[pl.ds(base, SCAT_R)], flatb_v)
        pltpu.sync_copy(vals_hbm.at[pl.ds(base, SCAT_R)], valb_v)
        pltpu.sync_copy(bitw_hbm.at[pl.ds(base, SCAT_R)], bitwb_v)
        pltpu.sync_copy(bitv_hbm.at[pl.ds(base, SCAT_R)], bitvb_v)
        for r in range(SCAT_R):
            for j in range(8):
                sl = pl.ds(j * 16, 16)
                flatb_v[r, sl] = flatb_v[r, sl] + coff
        for r in range(SCAT_R):
            pltpu.async_copy(valb_v.at[r], grid_hbm.at[flatb_v.at[r]], semg)
            pltpu.sync_copy(bitvb_v.at[r], bmap_sh.at[bitwb_v.at[r]],
                            add=True)
        for r in range(SCAT_R):
            pltpu.make_async_copy(valb_v.at[r], grid_hbm.at[flatb_v.at[r]],
                                  semg).wait()

    plsc.subcore_barrier()

    @pl.when(cid == 0)
    def _():
        pltpu.sync_copy(bmap_sh.at[pl.ds(sid * NBW_TILE, NBW_TILE)],
                        bmap_hbm.at[pl.ds(sid * NBW_TILE, NBW_TILE)])


def _sc_gather_kernel(flat_hbm, bmap_hbm, grid2_hbm, feats_hbm,
                      g_hbm, presw_hbm,
                      bmap_v, flatc_v, cadr_v, cdst_v, rowg_v, crid2_v,
                      cdst2_v, rowsb_v, presb_v):
    cid = lax.axis_index("c")
    sid = lax.axis_index("s")
    wid = cid * NS + sid
    coff16 = cid * GSZ16

    pltpu.sync_copy(bmap_hbm, bmap_v)

    iota16 = lax.iota(jnp.int32, 16)
    one16 = jnp.full((16,), 1, jnp.int32)
    i16x16 = iota16 * 16
    clampf = jnp.full((16,), CLAMPF, jnp.int32)

    @pl.loop(0, NBLK)
    def _(blk):
        vstart = wid * CHUNK + blk * BV
        vrow = vstart // BV
        pltpu.sync_copy(flat_hbm.at[pl.ds(vrow, 1)], flatc_v)

        off = jnp.int32(0)
        vs26 = vstart * NT
        for j in range(BV // 16):
            f = jnp.minimum(flatc_v[0, pl.ds(j * 16, 16)], clampf)
            pw = jnp.zeros((16,), jnp.int32)
            for oi in range(NT):
                adr = f + DELTA[oi]
                w = plsc.load_gather(bmap_v,
                                     [lax.shift_right_logical(adr, 5)])
                bit = lax.bitwise_and(
                    lax.shift_right_logical(w, lax.bitwise_and(adr, 31)),
                    one16)
                m = bit > 0
                pw = pw + lax.shift_left(bit, oi)
                ps = plsc.cumsum(bit)
                pos = (ps - bit) + off
                prow = lax.shift_right_logical(pos, 7)
                pcol = lax.bitwise_and(pos, 127)
                row16 = lax.shift_right_logical(adr, 4) + coff16
                dst4 = t26d4(vs26, j, oi, iota16) + lax.bitwise_and(adr, 15)
                plsc.store_scatter(cadr_v, [prow, pcol], row16, mask=m)
                plsc.store_scatter(cdst_v, [prow, pcol], dst4, mask=m)
                off = off + jnp.max(ps)
            presb_v[0, pl.ds(j * 16, 16)] = pw

        pltpu.sync_copy(presb_v, presw_hbm.at[pl.ds(vrow, 1)])

        lastrow = lax.shift_right_logical(off, 7)
        fillstart = lax.bitwise_and(off, 127)
        saferow = jnp.full((16,), SAFE_CELL // 16, jnp.int32) + coff16
        for j in range(8):
            lane = iota16 + (j * 16)
            fm = lane >= fillstart
            lr = jnp.broadcast_to(lastrow, (16,))
            plsc.store_scatter(cadr_v, [lr, lane], saferow, mask=fm)
            plsc.store_scatter(
                cdst_v, [lr, lane],
                lax.shift_left(jnp.full((16,), N * NT, jnp.int32) + iota16,
                               4), mask=fm)

        nch = lax.shift_right_logical(off + 127, 7)

        @pl.loop(0, NCH)
        def _(t):
            @pl.when(t < nch)
            def _():
                pltpu.sync_copy(grid2_hbm.at[cadr_v.at[t]], rowg_v)
                ts = jnp.broadcast_to(t, (16,))
                for v in range(8):
                    sl = pl.ds(v * 16, 16)
                    kv = iota16 + (v * 16)
                    d4 = plsc.load_gather(cdst_v, [ts, kv])
                    lan = lax.bitwise_and(d4, 15)
                    rid = plsc.load_gather(rowg_v, [kv, lan])
                    crid2_v[sl] = rid
                    cdst2_v[sl] = lax.shift_right_logical(d4, 4)
                pltpu.sync_copy(feats_hbm.at[crid2_v], rowsb_v)
                pltpu.sync_copy(rowsb_v, g_hbm.at[cdst2_v])


def t26d4(vs26, j, oi, iota16):
    return lax.shift_left(iota16 * NT + (vs26 + j * 16 * NT + oi), 4)


def _tc_kernel(g_ref, feats_ref, presw_ref, w26_ref, wc_ref, b_ref, o_ref,
               *, bn):
    g = g_ref[...]
    pw = presw_ref[...]
    lane = lax.broadcasted_iota(jnp.int32, (1, NT * CIN), 1)
    sh = lane // CIN
    msk = jnp.right_shift(pw, sh) & 1
    gm = (g * msk.astype(jnp.float32)).astype(jnp.bfloat16)
    acc = jnp.dot(gm, w26_ref[...].astype(jnp.bfloat16),
                  preferred_element_type=jnp.float32)
    acc = acc + jnp.dot(feats_ref[...].astype(jnp.bfloat16),
                        wc_ref[...].astype(jnp.bfloat16),
                        preferred_element_type=jnp.float32)
    o_ref[...] = acc + b_ref[...]


@jax.jit
def kernel(feats, coords, weight, bias):
    n = feats.shape[0]
    fg = (((coords[:, 0] + 1) * GDIM + (coords[:, 1] + 1)) * GDIM
          + (coords[:, 2] + 1)).astype(jnp.int32)
    pad_flat = jnp.broadcast_to(fg[0], (NP - n,))
    pad_flat = pad_flat.at[0].set(SAFE_CELL)
    flat_pad = jnp.concatenate([fg, pad_flat]).reshape(NROWS, 128)
    vals = jnp.concatenate(
        [jnp.arange(n, dtype=jnp.int32),
         jnp.zeros((NP - n,), jnp.int32)]).reshape(NROWS, 128)
    bitw = flat_pad >> 5
    bitv = jnp.where(
        (jnp.arange(NP, dtype=jnp.int32) < n).reshape(NROWS, 128),
        jnp.int32(1) << (flat_pad & 31), 0)

    mesh = plsc.VectorSubcoreMesh(core_axis_name="c", subcore_axis_name="s")
    cp = pltpu.CompilerParams(needs_layout_passes=False,
                              use_tc_tiling_on_sc=False)
    build = pl.kernel(
        _sc_build_kernel,
        out_type=(
            jax.ShapeDtypeStruct((NC * GSZ,), jnp.int32),
            jax.ShapeDtypeStruct((NBW,), jnp.int32),
        ),
        mesh=mesh,
        compiler_params=cp,
        scratch_types=[
            pltpu.VMEM_SHARED((NBW,), jnp.int32),
            pltpu.VMEM((NBW_TILE,), jnp.int32),
            pltpu.VMEM((SCAT_R, 128), jnp.int32),
            pltpu.VMEM((SCAT_R, 128), jnp.int32),
            pltpu.VMEM((SCAT_R, 128), jnp.int32),
            pltpu.VMEM((SCAT_R, 128), jnp.int32),
            pltpu.SemaphoreType.DMA,
        ],
    )
    grid1d, bmap = build(flat_pad, vals, bitw, bitv)
    grid2 = grid1d.reshape(NC * GSZ16, 16)

    gather = pl.kernel(
        _sc_gather_kernel,
        out_type=(
            jax.ShapeDtypeStruct((NP * NT, CIN), jnp.float32),
            jax.ShapeDtypeStruct((NROWS, 128), jnp.int32),
        ),
        mesh=mesh,
        compiler_params=cp,
        scratch_types=[
            pltpu.VMEM((NBW,), jnp.int32),
            pltpu.VMEM((1, 128), jnp.int32),
            pltpu.VMEM((NCH + 1, 128), jnp.int32),
            pltpu.VMEM((NCH + 1, 128), jnp.int32),
            pltpu.VMEM((128, 16), jnp.int32),
            pltpu.VMEM((128,), jnp.int32),
            pltpu.VMEM((128,), jnp.int32),
            pltpu.VMEM((128, CIN), jnp.float32),
            pltpu.VMEM((1, 128), jnp.int32),
        ],
    )
    g, presw = gather(flat_pad, bmap, grid2, feats)

    g3 = g.reshape(NP, NT * CIN)
    presw2 = presw.reshape(NP, 1)
    feats_pad = jnp.zeros((NP, CIN), jnp.float32).at[:n].set(feats)

    w = jnp.transpose(weight, (1, 2, 3, 4, 0)).reshape(27, CIN, COUT)
    keep = [i for i in range(27) if i != 13]
    w26 = w[jnp.array(keep)].reshape(NT * CIN, COUT)
    wc = w[13]
    b2 = bias.reshape(1, COUT)

    bn = 2048
    out = pl.pallas_call(
        functools.partial(_tc_kernel, bn=bn),
        grid=(NP // bn,),
        in_specs=[
            pl.BlockSpec((bn, NT * CIN), lambda i: (i, 0)),
            pl.BlockSpec((bn, CIN), lambda i: (i, 0)),
            pl.BlockSpec((bn, 1), lambda i: (i, 0)),
            pl.BlockSpec((NT * CIN, COUT), lambda i: (0, 0)),
            pl.BlockSpec((CIN, COUT), lambda i: (0, 0)),
            pl.BlockSpec((1, COUT), lambda i: (0, 0)),
        ],
        out_specs=pl.BlockSpec((bn, COUT), lambda i: (i, 0)),
        out_shape=jax.ShapeDtypeStruct((NP, COUT), jnp.float32),
    )(g3, feats_pad, presw2, w26, wc, b2)
    return out[:n]


# speedup vs baseline: 9.5741x; 1.0343x over previous
import functools

import jax
import jax.numpy as jnp
from jax import lax
from jax.experimental import pallas as pl
from jax.experimental.pallas import tpu as pltpu
from jax.experimental.pallas import tpu_sc as plsc

GRID = 128
N = 100000
CIN = 16
COUT = 16
K = 3

NC = 2
NS = 16
NW = NC * NS
CHUNK = 3200
NP = NW * CHUNK
NROWS = NP // 128
BV = 128
NBLK = CHUNK // BV
NT = 26

GDIM = GRID + 2
NCELL = GDIM ** 3
SAFE_CELL = 2197008
GSZ = 2197120
GSZ16 = GSZ // 16
CLAMPF = 2179968
NBW = 68864
NBW_TILE = NBW // NS
NCH = BV * NT // 128
SCAT_R = 5

OFFS = [(dz, dy, dx) for dz in range(K) for dy in range(K) for dx in range(K)
        if (dz, dy, dx) != (1, 1, 1)]
DELTA = [
    (dz - 1) * GDIM * GDIM + (dy - 1) * GDIM + (dx - 1) for dz, dy, dx in OFFS
]


def _sc_build_kernel(flat_hbm, vals_hbm, bitw_hbm, bitv_hbm,
                     grid_hbm, bmap_hbm,
                     bmap_sh, zero_v, flatb_v, valb_v, bitwb_v, bitvb_v,
                     semg):
    cid = lax.axis_index("c")
    sid = lax.axis_index("s")
    wid = cid * NS + sid

    @pl.loop(0, NBW_TILE, step=16)
    def _(i):
        zero_v[pl.ds(i, 16)] = jnp.zeros((16,), jnp.int32)

    pltpu.sync_copy(zero_v, bmap_sh.at[pl.ds(sid * NBW_TILE, NBW_TILE)])
    plsc.subcore_barrier()

    rows_per_tile = NROWS // NW

    @pl.loop(0, rows_per_tile, step=SCAT_R)
    def _(i):
        base = wid * rows_per_tile + i
        pltpu.sync_copy(flat_hbm.at[pl.ds(base, SCAT_R)], flatb_v)
        pltpu.sync_copy(vals_hbm.at[pl.ds(base, SCAT_R)], valb_v)
        pltpu.sync_copy(bitw_hbm.at[pl.ds(base, SCAT_R)], bitwb_v)
        pltpu.sync_copy(bitv_hbm.at[pl.ds(base, SCAT_R)], bitvb_v)
        for r in range(SCAT_R):
            pltpu.async_copy(valb_v.at[r], grid_hbm.at[flatb_v.at[r]], semg)
            pltpu.sync_copy(bitvb_v.at[r], bmap_sh.at[bitwb_v.at[r]],
                            add=True)
        for r in range(SCAT_R):
            pltpu.make_async_copy(valb_v.at[r], grid_hbm.at[flatb_v.at[r]],
                                  semg).wait()

    plsc.subcore_barrier()

    pltpu.sync_copy(bmap_sh.at[pl.ds(sid * NBW_TILE, NBW_TILE)],
                    bmap_hbm.at[cid].at[pl.ds(sid * NBW_TILE, NBW_TILE)])


def _sc_gather_kernel(flat_hbm, bmap_hbm, grid2_hbm, feats_hbm,
                      g_hbm, presw_hbm,
                      bmap_v, tmp_v, flatall_v, cadr_v, cdst_v, rowg_v,
                      crid2_v, cdst2_v, rowsb_v, presall_v):
    cid = lax.axis_index("c")
    sid = lax.axis_index("s")
    wid = cid * NS + sid

    pltpu.sync_copy(bmap_hbm.at[0], bmap_v)

    @pl.loop(0, NBW, step=NBW_TILE)
    def _(i):
        pltpu.sync_copy(bmap_hbm.at[1].at[pl.ds(i, NBW_TILE)], tmp_v)
        for k in range(0, NBW_TILE, 16):
            sl = pl.ds(i + k, 16)
            bmap_v[sl] = lax.bitwise_or(bmap_v[sl], tmp_v[pl.ds(k, 16)])

    pltpu.sync_copy(flat_hbm.at[pl.ds(wid * NBLK, NBLK)], flatall_v)

    iota16 = lax.iota(jnp.int32, 16)
    one16 = jnp.full((16,), 1, jnp.int32)
    clampf = jnp.full((16,), CLAMPF, jnp.int32)

    @pl.loop(0, NBLK)
    def _(blk):
        vstart = wid * CHUNK + blk * BV
        blkv = jnp.broadcast_to(blk, (16,))

        off = jnp.int32(0)
        vs26 = vstart * NT
        for j in range(BV // 16):
            f = jnp.minimum(
                plsc.load_gather(flatall_v, [blkv, iota16 + j * 16]),
                clampf)
            pw = jnp.zeros((16,), jnp.int32)
            for oi in range(NT):
                adr = f + DELTA[oi]
                w = plsc.load_gather(bmap_v,
                                     [lax.shift_right_logical(adr, 5)])
                bit = lax.bitwise_and(
                    lax.shift_right_logical(w, lax.bitwise_and(adr, 31)),
                    one16)
                m = bit > 0
                pw = pw + lax.shift_left(bit, oi)
                ps = plsc.cumsum(bit)
                pos = (ps - bit) + off
                prow = lax.shift_right_logical(pos, 7)
                pcol = lax.bitwise_and(pos, 127)
                row16 = lax.shift_right_logical(adr, 4)
                dst4 = t26d4(vs26, j, oi, iota16) + lax.bitwise_and(adr, 15)
                plsc.store_scatter(cadr_v, [prow, pcol], row16, mask=m)
                plsc.store_scatter(cdst_v, [prow, pcol], dst4, mask=m)
                off = off + jnp.max(ps)
            plsc.store_scatter(presall_v, [blkv, iota16 + j * 16], pw)

        lastrow = lax.shift_right_logical(off, 7)
        fillstart = lax.bitwise_and(off, 127)
        saferow = jnp.full((16,), SAFE_CELL // 16, jnp.int32)
        for j in range(8):
            lane = iota16 + (j * 16)
            fm = lane >= fillstart
            lr = jnp.broadcast_to(lastrow, (16,))
            plsc.store_scatter(cadr_v, [lr, lane], saferow, mask=fm)
            plsc.store_scatter(
                cdst_v, [lr, lane],
                lax.shift_left(jnp.full((16,), N * NT, jnp.int32) + iota16,
                               4), mask=fm)

        nch = lax.shift_right_logical(off + 127, 7)

        @pl.loop(0, NCH)
        def _(t):
            @pl.when(t < nch)
            def _():
                pltpu.sync_copy(grid2_hbm.at[cadr_v.at[t]], rowg_v)
                ts = jnp.broadcast_to(t, (16,))
                for v in range(8):
                    sl = pl.ds(v * 16, 16)
                    kv = iota16 + (v * 16)
                    d4 = plsc.load_gather(cdst_v, [ts, kv])
                    lan = lax.bitwise_and(d4, 15)
                    rid = plsc.load_gather(rowg_v, [kv, lan])
                    crid2_v[sl] = rid
                    cdst2_v[sl] = lax.shift_right_logical(d4, 4)
                pltpu.sync_copy(feats_hbm.at[crid2_v], rowsb_v)
                pltpu.sync_copy(rowsb_v, g_hbm.at[cdst2_v])

    pltpu.sync_copy(presall_v, presw_hbm.at[pl.ds(wid * NBLK, NBLK)])


def t26d4(vs26, j, oi, iota16):
    return lax.shift_left(iota16 * NT + (vs26 + j * 16 * NT + oi), 4)


def _tc_kernel(g_ref, feats_ref, presw_ref, w26_ref, wc_ref, b_ref, o_ref,
               *, bn):
    g = g_ref[...]
    pw = presw_ref[...]
    lane = lax.broadcasted_iota(jnp.int32, (1, NT * CIN), 1)
    sh = lane // CIN
    msk = jnp.right_shift(pw, sh) & 1
    gm = (g * msk.astype(jnp.float32)).astype(jnp.bfloat16)
    acc = jnp.dot(gm, w26_ref[...].astype(jnp.bfloat16),
                  preferred_element_type=jnp.float32)
    acc = acc + jnp.dot(feats_ref[...].astype(jnp.bfloat16),
                        wc_ref[...].astype(jnp.bfloat16),
                        preferred_element_type=jnp.float32)
    o_ref[...] = acc + b_ref[...]


@jax.jit
def kernel(feats, coords, weight, bias):
    n = feats.shape[0]
    fg = (((coords[:, 0] + 1) * GDIM + (coords[:, 1] + 1)) * GDIM
          + (coords[:, 2] + 1)).astype(jnp.int32)
    pad_flat = jnp.broadcast_to(fg[0], (NP - n,))
    pad_flat = pad_flat.at[0].set(SAFE_CELL)
    flat_pad = jnp.concatenate([fg, pad_flat]).reshape(NROWS, 128)
    vals = jnp.concatenate(
        [jnp.arange(n, dtype=jnp.int32),
         jnp.zeros((NP - n,), jnp.int32)]).reshape(NROWS, 128)
    bitw = flat_pad >> 5
    bitv = jnp.where(
        (jnp.arange(NP, dtype=jnp.int32) < n).reshape(NROWS, 128),
        jnp.int32(1) << (flat_pad & 31), 0)

    mesh = plsc.VectorSubcoreMesh(core_axis_name="c", subcore_axis_name="s")
    cp = pltpu.CompilerParams(needs_layout_passes=False,
                              use_tc_tiling_on_sc=False)
    build = pl.kernel(
        _sc_build_kernel,
        out_type=(
            jax.ShapeDtypeStruct((GSZ,), jnp.int32),
            jax.ShapeDtypeStruct((NC, NBW), jnp.int32),
        ),
        mesh=mesh,
        compiler_params=cp,
        scratch_types=[
            pltpu.VMEM_SHARED((NBW,), jnp.int32),
            pltpu.VMEM((NBW_TILE,), jnp.int32),
            pltpu.VMEM((SCAT_R, 128), jnp.int32),
            pltpu.VMEM((SCAT_R, 128), jnp.int32),
            pltpu.VMEM((SCAT_R, 128), jnp.int32),
            pltpu.VMEM((SCAT_R, 128), jnp.int32),
            pltpu.SemaphoreType.DMA,
        ],
    )
    grid1d, bmap = build(flat_pad, vals, bitw, bitv)
    grid2 = grid1d.reshape(GSZ16, 16)

    gather = pl.kernel(
        _sc_gather_kernel,
        out_type=(
            jax.ShapeDtypeStruct((NP * NT, CIN), jnp.float32),
            jax.ShapeDtypeStruct((NROWS, 128), jnp.int32),
        ),
        mesh=mesh,
        compiler_params=cp,
        scratch_types=[
            pltpu.VMEM((NBW,), jnp.int32),
            pltpu.VMEM((NBW_TILE,), jnp.int32),
            pltpu.VMEM((NBLK, 128), jnp.int32),
            pltpu.VMEM((NCH + 1, 128), jnp.int32),
            pltpu.VMEM((NCH + 1, 128), jnp.int32),
            pltpu.VMEM((128, 16), jnp.int32),
            pltpu.VMEM((128,), jnp.int32),
            pltpu.VMEM((128,), jnp.int32),
            pltpu.VMEM((128, CIN), jnp.float32),
            pltpu.VMEM((NBLK, 128), jnp.int32),
        ],
    )
    g, presw = gather(flat_pad, bmap, grid2, feats)

    g3 = g.reshape(NP, NT * CIN)
    presw2 = presw.reshape(NP, 1)
    feats_pad = jnp.zeros((NP, CIN), jnp.float32).at[:n].set(feats)

    w = jnp.transpose(weight, (1, 2, 3, 4, 0)).reshape(27, CIN, COUT)
    keep = [i for i in range(27) if i != 13]
    w26 = w[jnp.array(keep)].reshape(NT * CIN, COUT)
    wc = w[13]
    b2 = bias.reshape(1, COUT)

    bn = 2048
    out = pl.pallas_call(
        functools.partial(_tc_kernel, bn=bn),
        grid=(NP // bn,),
        in_specs=[
            pl.BlockSpec((bn, NT * CIN), lambda i: (i, 0)),
            pl.BlockSpec((bn, CIN), lambda i: (i, 0)),
            pl.BlockSpec((bn, 1), lambda i: (i, 0)),
            pl.BlockSpec((NT * CIN, COUT), lambda i: (0, 0)),
            pl.BlockSpec((CIN, COUT), lambda i: (0, 0)),
            pl.BlockSpec((1, COUT), lambda i: (0, 0)),
        ],
        out_specs=pl.BlockSpec((bn, COUT), lambda i: (i, 0)),
        out_shape=jax.ShapeDtypeStruct((NP, COUT), jnp.float32),
    )(g3, feats_pad, presw2, w26, wc, b2)
    return out[:n]


# speedup vs baseline: 9.6527x; 1.0082x over previous
import functools

import jax
import jax.numpy as jnp
from jax import lax
from jax.experimental import pallas as pl
from jax.experimental.pallas import tpu as pltpu
from jax.experimental.pallas import tpu_sc as plsc

GRID = 128
N = 100000
CIN = 16
COUT = 16
K = 3

NC = 2
NS = 16
NW = NC * NS
CHUNK = 3200
NP = NW * CHUNK
NROWS = NP // 128
BV = 128
NBLK = CHUNK // BV
NT = 26

GDIM = GRID + 2
NCELL = GDIM ** 3
SAFE_CELL = 2197008
GSZ = 2197120
GSZ16 = GSZ // 16
CLAMPF = 2179968
NBW = 68864
NBW_TILE = NBW // NS
NCH = BV * NT // 128
SCAT_R = 5

OFFS = [(dz, dy, dx) for dz in range(K) for dy in range(K) for dx in range(K)
        if (dz, dy, dx) != (1, 1, 1)]
DELTA = [
    (dz - 1) * GDIM * GDIM + (dy - 1) * GDIM + (dx - 1) for dz, dy, dx in OFFS
]


def _sc_build_kernel(flat_hbm, vals_hbm, bitw_hbm, bitv_hbm,
                     grid_hbm, bmap_hbm,
                     bmap_sh, zero_v, flatb_v, valb_v, bitwb_v, bitvb_v,
                     semg):
    cid = lax.axis_index("c")
    sid = lax.axis_index("s")
    wid = cid * NS + sid

    @pl.loop(0, NBW_TILE, step=16)
    def _(i):
        zero_v[pl.ds(i, 16)] = jnp.zeros((16,), jnp.int32)

    pltpu.sync_copy(zero_v, bmap_sh.at[pl.ds(sid * NBW_TILE, NBW_TILE)])
    plsc.subcore_barrier()

    rows_per_tile = NROWS // NW

    @pl.loop(0, rows_per_tile, step=SCAT_R)
    def _(i):
        base = wid * rows_per_tile + i
        pltpu.sync_copy(flat_hbm.at[pl.ds(base, SCAT_R)], flatb_v)
        pltpu.sync_copy(vals_hbm.at[pl.ds(base, SCAT_R)], valb_v)
        pltpu.sync_copy(bitw_hbm.at[pl.ds(base, SCAT_R)], bitwb_v)
        pltpu.sync_copy(bitv_hbm.at[pl.ds(base, SCAT_R)], bitvb_v)
        for r in range(SCAT_R):
            pltpu.async_copy(valb_v.at[r], grid_hbm.at[flatb_v.at[r]], semg)
            pltpu.sync_copy(bitvb_v.at[r], bmap_sh.at[bitwb_v.at[r]],
                            add=True)
        for r in range(SCAT_R):
            pltpu.make_async_copy(valb_v.at[r], grid_hbm.at[flatb_v.at[r]],
                                  semg).wait()

    plsc.subcore_barrier()

    pltpu.sync_copy(bmap_sh.at[pl.ds(sid * NBW_TILE, NBW_TILE)],
                    bmap_hbm.at[cid].at[pl.ds(sid * NBW_TILE, NBW_TILE)])


def _sc_gather_kernel(flat_hbm, bmap_hbm, grid2_hbm, feats_hbm,
                      g_hbm, presw_hbm,
                      bmap_v, tmp_v, flatall_v, cadr_v, cdst_v, rowg_v,
                      crid2_v, cdst2_v, rowsb_v, presall_v,
                      semc0, semc1, sems0, sems1):
    cid = lax.axis_index("c")
    sid = lax.axis_index("s")
    wid = cid * NS + sid

    pltpu.sync_copy(bmap_hbm.at[0], bmap_v)

    @pl.loop(0, NBW, step=NBW_TILE)
    def _(i):
        pltpu.sync_copy(bmap_hbm.at[1].at[pl.ds(i, NBW_TILE)], tmp_v)
        for k in range(0, NBW_TILE, 16):
            sl = pl.ds(i + k, 16)
            bmap_v[sl] = lax.bitwise_or(bmap_v[sl], tmp_v[pl.ds(k, 16)])

    pltpu.sync_copy(flat_hbm.at[pl.ds(wid * NBLK, NBLK)], flatall_v)

    iota16 = lax.iota(jnp.int32, 16)
    one16 = jnp.full((16,), 1, jnp.int32)
    clampf = jnp.full((16,), CLAMPF, jnp.int32)

    @pl.loop(0, NBLK)
    def _(blk):
        vstart = wid * CHUNK + blk * BV
        blkv = jnp.broadcast_to(blk, (16,))

        off = jnp.int32(0)
        vs26 = vstart * NT
        for j in range(BV // 16):
            f = jnp.minimum(
                plsc.load_gather(flatall_v, [blkv, iota16 + j * 16]),
                clampf)
            pw = jnp.zeros((16,), jnp.int32)
            for oi in range(NT):
                adr = f + DELTA[oi]
                w = plsc.load_gather(bmap_v,
                                     [lax.shift_right_logical(adr, 5)])
                bit = lax.bitwise_and(
                    lax.shift_right_logical(w, lax.bitwise_and(adr, 31)),
                    one16)
                m = bit > 0
                pw = pw + lax.shift_left(bit, oi)
                ps = plsc.cumsum(bit)
                pos = (ps - bit) + off
                prow = lax.shift_right_logical(pos, 7)
                pcol = lax.bitwise_and(pos, 127)
                row16 = lax.shift_right_logical(adr, 4)
                dst4 = t26d4(vs26, j, oi, iota16) + lax.bitwise_and(adr, 15)
                plsc.store_scatter(cadr_v, [prow, pcol], row16, mask=m)
                plsc.store_scatter(cdst_v, [prow, pcol], dst4, mask=m)
                off = off + jnp.max(ps)
            plsc.store_scatter(presall_v, [blkv, iota16 + j * 16], pw)

        lastrow = lax.shift_right_logical(off, 7)
        fillstart = lax.bitwise_and(off, 127)
        saferow = jnp.full((16,), SAFE_CELL // 16, jnp.int32)
        for j in range(8):
            lane = iota16 + (j * 16)
            fm = lane >= fillstart
            lr = jnp.broadcast_to(lastrow, (16,))
            plsc.store_scatter(cadr_v, [lr, lane], saferow, mask=fm)
            plsc.store_scatter(
                cdst_v, [lr, lane],
                lax.shift_left(jnp.full((16,), N * NT, jnp.int32) + iota16,
                               4), mask=fm)

        nch = lax.shift_right_logical(off + 127, 7)

        semc = (semc0, semc1)
        sems = (sems0, sems1)

        def fire_grid(t, s):
            pltpu.async_copy(grid2_hbm.at[cadr_v.at[t]], rowg_v.at[s],
                             semc[s])

        def wait_grid(t, s):
            pltpu.make_async_copy(grid2_hbm.at[cadr_v.at[t]], rowg_v.at[s],
                                  semc[s]).wait()

        def wait_scat(s):
            pltpu.make_async_copy(rowsb_v.at[s], g_hbm.at[cdst2_v.at[s]],
                                  sems[s]).wait()

        @pl.when(nch > 0)
        def _():
            fire_grid(0, 0)

        @pl.loop(0, (NCH + 2) // 2)
        def _(p):
            for s in (0, 1):
                t = p * 2 + s

                @pl.when(t < nch)
                def _():
                    @pl.when(t + 1 < nch)
                    def _():
                        fire_grid(t + 1, 1 - s)

                    wait_grid(t, s)

                    @pl.when(t >= 2)
                    def _():
                        wait_scat(s)

                    ts = jnp.broadcast_to(t, (16,))
                    for v in range(8):
                        sl = pl.ds(v * 16, 16)
                        kv = iota16 + (v * 16)
                        d4 = plsc.load_gather(cdst_v, [ts, kv])
                        lan = lax.bitwise_and(d4, 15)
                        rid = plsc.load_gather(rowg_v.at[s], [kv, lan])
                        crid2_v[s, sl] = rid
                        cdst2_v[s, sl] = lax.shift_right_logical(d4, 4)
                    pltpu.sync_copy(feats_hbm.at[crid2_v.at[s]],
                                    rowsb_v.at[s])
                    pltpu.async_copy(rowsb_v.at[s], g_hbm.at[cdst2_v.at[s]],
                                     sems[s])

        @pl.when(nch > 0)
        def _():
            wait_scat(0)

        @pl.when(nch > 1)
        def _():
            wait_scat(1)

    pltpu.sync_copy(presall_v, presw_hbm.at[pl.ds(wid * NBLK, NBLK)])


def t26d4(vs26, j, oi, iota16):
    return lax.shift_left(iota16 * NT + (vs26 + j * 16 * NT + oi), 4)


def _tc_kernel(g_ref, feats_ref, presw_ref, w26_ref, wc_ref, b_ref, o_ref,
               *, bn):
    g = g_ref[...]
    pw = presw_ref[...]
    lane = lax.broadcasted_iota(jnp.int32, (1, NT * CIN), 1)
    sh = lane // CIN
    msk = jnp.right_shift(pw, sh) & 1
    gm = (g * msk.astype(jnp.float32)).astype(jnp.bfloat16)
    acc = jnp.dot(gm, w26_ref[...].astype(jnp.bfloat16),
                  preferred_element_type=jnp.float32)
    acc = acc + jnp.dot(feats_ref[...].astype(jnp.bfloat16),
                        wc_ref[...].astype(jnp.bfloat16),
                        preferred_element_type=jnp.float32)
    o_ref[...] = acc + b_ref[...]


@jax.jit
def kernel(feats, coords, weight, bias):
    n = feats.shape[0]
    fg = (((coords[:, 0] + 1) * GDIM + (coords[:, 1] + 1)) * GDIM
          + (coords[:, 2] + 1)).astype(jnp.int32)
    pad_flat = jnp.broadcast_to(fg[0], (NP - n,))
    pad_flat = pad_flat.at[0].set(SAFE_CELL)
    flat_pad = jnp.concatenate([fg, pad_flat]).reshape(NROWS, 128)
    vals = jnp.concatenate(
        [jnp.arange(n, dtype=jnp.int32),
         jnp.zeros((NP - n,), jnp.int32)]).reshape(NROWS, 128)
    bitw = flat_pad >> 5
    bitv = jnp.where(
        (jnp.arange(NP, dtype=jnp.int32) < n).reshape(NROWS, 128),
        jnp.int32(1) << (flat_pad & 31), 0)

    mesh = plsc.VectorSubcoreMesh(core_axis_name="c", subcore_axis_name="s")
    cp = pltpu.CompilerParams(needs_layout_passes=False,
                              use_tc_tiling_on_sc=False)
    build = pl.kernel(
        _sc_build_kernel,
        out_type=(
            jax.ShapeDtypeStruct((GSZ,), jnp.int32),
            jax.ShapeDtypeStruct((NC, NBW), jnp.int32),
        ),
        mesh=mesh,
        compiler_params=cp,
        scratch_types=[
            pltpu.VMEM_SHARED((NBW,), jnp.int32),
            pltpu.VMEM((NBW_TILE,), jnp.int32),
            pltpu.VMEM((SCAT_R, 128), jnp.int32),
            pltpu.VMEM((SCAT_R, 128), jnp.int32),
            pltpu.VMEM((SCAT_R, 128), jnp.int32),
            pltpu.VMEM((SCAT_R, 128), jnp.int32),
            pltpu.SemaphoreType.DMA,
        ],
    )
    grid1d, bmap = build(flat_pad, vals, bitw, bitv)
    grid2 = grid1d.reshape(GSZ16, 16)

    gather = pl.kernel(
        _sc_gather_kernel,
        out_type=(
            jax.ShapeDtypeStruct((NP * NT, CIN), jnp.float32),
            jax.ShapeDtypeStruct((NROWS, 128), jnp.int32),
        ),
        mesh=mesh,
        compiler_params=cp,
        scratch_types=[
            pltpu.VMEM((NBW,), jnp.int32),
            pltpu.VMEM((NBW_TILE,), jnp.int32),
            pltpu.VMEM((NBLK, 128), jnp.int32),
            pltpu.VMEM((NCH + 1, 128), jnp.int32),
            pltpu.VMEM((NCH + 1, 128), jnp.int32),
            pltpu.VMEM((2, 128, 16), jnp.int32),
            pltpu.VMEM((2, 128), jnp.int32),
            pltpu.VMEM((2, 128), jnp.int32),
            pltpu.VMEM((2, 128, CIN), jnp.float32),
            pltpu.VMEM((NBLK, 128), jnp.int32),
            pltpu.SemaphoreType.DMA,
            pltpu.SemaphoreType.DMA,
            pltpu.SemaphoreType.DMA,
            pltpu.SemaphoreType.DMA,
        ],
    )
    g, presw = gather(flat_pad, bmap, grid2, feats)

    g3 = g.reshape(NP, NT * CIN)
    presw2 = presw.reshape(NP, 1)
    feats_pad = jnp.zeros((NP, CIN), jnp.float32).at[:n].set(feats)

    w = jnp.transpose(weight, (1, 2, 3, 4, 0)).reshape(27, CIN, COUT)
    keep = [i for i in range(27) if i != 13]
    w26 = w[jnp.array(keep)].reshape(NT * CIN, COUT)
    wc = w[13]
    b2 = bias.reshape(1, COUT)

    bn = 2048
    out = pl.pallas_call(
        functools.partial(_tc_kernel, bn=bn),
        grid=(NP // bn,),
        in_specs=[
            pl.BlockSpec((bn, NT * CIN), lambda i: (i, 0)),
            pl.BlockSpec((bn, CIN), lambda i: (i, 0)),
            pl.BlockSpec((bn, 1), lambda i: (i, 0)),
            pl.BlockSpec((NT * CIN, COUT), lambda i: (0, 0)),
            pl.BlockSpec((CIN, COUT), lambda i: (0, 0)),
            pl.BlockSpec((1, COUT), lambda i: (0, 0)),
        ],
        out_specs=pl.BlockSpec((bn, COUT), lambda i: (i, 0)),
        out_shape=jax.ShapeDtypeStruct((NP, COUT), jnp.float32),
    )(g3, feats_pad, presw2, w26, wc, b2)
    return out[:n]
